# baseline (speedup 1.0000x reference)
import jax
import jax.numpy as jnp
from jax.experimental import pallas as pl
from jax.experimental.pallas import tpu as pltpu

BN_EPS = 1e-5
_NBUF = 4


def _fused_kernel(x_hbm, onehot_ref, p256_ref, p64_ref, p128_ref, out_ref,
                  h1_ref, h2_ref, s1_ref, s2_ref):
    step = pl.program_id(0)
    nblk = pl.num_programs(0) // 3
    blk = x_hbm.shape[0]
    b_total = h1_ref.shape[0]
    inv_b = 1.0 / b_total
    h0 = h1_ref.shape[1]
    h1w = h2_ref.shape[1]

    @pl.when(step < nblk)
    def _phase_a():
        xb = x_hbm[...].astype(jnp.bfloat16)
        w1 = p256_ref[0:512, :].astype(jnp.bfloat16)
        h1 = jnp.dot(xb, w1, preferred_element_type=jnp.float32)
        h1_ref[pl.ds(step * blk, blk), :] = h1.astype(jnp.bfloat16)

        @pl.when(step == 0)
        def _():
            s1_ref[...] = jnp.zeros_like(s1_ref)

        s1_ref[...] += jnp.stack([jnp.sum(h1, axis=0),
                                  jnp.sum(h1 * h1, axis=0)])

    @pl.when((step >= nblk) & (step < 2 * nblk))
    def _phase_b():
        i = step - nblk
        totals = s1_ref[...]
        mean = totals[0:1, :] * inv_b
        var = totals[1:2, :] * inv_b - mean * mean
        gamma = p256_ref[512:513, :]
        beta = p256_ref[513:514, :]
        scale = gamma * jax.lax.rsqrt(var + BN_EPS)
        shift = beta - mean * scale
        h1 = h1_ref[pl.ds(i * blk, blk), :].astype(jnp.float32)
        h1n = jnp.maximum(h1 * scale + shift, 0.0)
        w2 = p64_ref[0:256, :].astype(jnp.bfloat16)
        h2 = jnp.dot(h1n.astype(jnp.bfloat16), w2,
                     preferred_element_type=jnp.float32)
        h2_ref[pl.ds(i * blk, blk), :] = h2.astype(jnp.bfloat16)

        @pl.when(i == 0)
        def _():
            s2_ref[...] = jnp.zeros_like(s2_ref)

        s2_ref[...] += jnp.stack([jnp.sum(h2, axis=0),
                                  jnp.sum(h2 * h2, axis=0)])

    @pl.when(step >= 2 * nblk)
    def _phase_c():
        i = step - 2 * nblk
        totals = s2_ref[...]
        mean = totals[0:1, :] * inv_b
        var = totals[1:2, :] * inv_b - mean * mean
        gamma = p64_ref[256:257, :]
        beta = p64_ref[257:258, :]
        scale = gamma * jax.lax.rsqrt(var + BN_EPS)
        shift = beta - mean * scale
        h2 = h2_ref[pl.ds(i * blk, blk), :].astype(jnp.float32)
        h2n = jnp.maximum(h2 * scale + shift, 0.0)
        oh = onehot_ref[pl.ds(i * blk, blk), :]
        w3a = p128_ref[0:64, :].astype(jnp.bfloat16)
        w3b = p128_ref[64:80, :].astype(jnp.bfloat16)
        b3 = p128_ref[80:81, :]
        out_dim = out_ref.shape[1]
        logits = (jnp.dot(h2n.astype(jnp.bfloat16), w3a,
                          preferred_element_type=jnp.float32)
                  + jnp.dot(oh.astype(jnp.bfloat16), w3b,
                            preferred_element_type=jnp.float32)
                  + b3)[:, :out_dim]
        out_ref[...] = jax.nn.sigmoid(logits)


def kernel(latent_vec, class_onehot, w1, bn1, w2, bn2, w3a_pad, w3b_pad,
           b3_pad, output_dim=3):
    B, latent_dim = latent_vec.shape
    H0 = w1.shape[1]
    H1 = w2.shape[1]
    OUTP = w3a_pad.shape[1]
    C = class_onehot.shape[1]

    blk = 2048 if B % 2048 == 0 else B
    nblk = B // blk
    nsteps = 3 * nblk

    p256 = jnp.concatenate([w1, bn1], axis=0)
    p64 = jnp.concatenate([w2, bn2], axis=0)
    p128 = jnp.concatenate([w3a_pad, w3b_pad, b3_pad], axis=0)

    flops = (2 * B * (latent_dim * H0 + H0 * H1 + H1 * OUTP + C * OUTP)
             + 12 * B * (H0 + H1))
    bytes_accessed = (B * latent_dim * 4 + B * C * 4
                      + (514 * H0 + 258 * H1 + 81 * OUTP) * 4
                      + B * output_dim * 4)

    grid_spec = pltpu.PrefetchScalarGridSpec(
        num_scalar_prefetch=0,
        grid=(nsteps,),
        in_specs=[
            pl.BlockSpec((blk, latent_dim),
                         lambda s, n=nblk: (jnp.minimum(s, n - 1), 0)),
            pl.BlockSpec((B, C), lambda s: (0, 0)),
            pl.BlockSpec(memory_space=pltpu.MemorySpace.VMEM),
            pl.BlockSpec(memory_space=pltpu.MemorySpace.VMEM),
            pl.BlockSpec(memory_space=pltpu.MemorySpace.VMEM),
        ],
        out_specs=pl.BlockSpec(
            (blk, output_dim),
            lambda s, n=nblk: (jnp.maximum(s - 2 * n, 0), 0)),
        scratch_shapes=[
            pltpu.VMEM((B, H0), jnp.bfloat16),
            pltpu.VMEM((B, H1), jnp.bfloat16),
            pltpu.VMEM((2, H0), jnp.float32),
            pltpu.VMEM((2, H1), jnp.float32),
        ],
    )

    return pl.pallas_call(
        _fused_kernel,
        out_shape=jax.ShapeDtypeStruct((B, output_dim), jnp.float32),
        grid_spec=grid_spec,
        compiler_params=pltpu.CompilerParams(
            dimension_semantics=("arbitrary",),
            vmem_limit_bytes=56 * 1024 * 1024),
        cost_estimate=pl.CostEstimate(
            flops=flops,
            transcendentals=B * output_dim + H0 + H1,
            bytes_accessed=bytes_accessed),
    )(latent_vec, class_onehot, p256, p64, p128)


# speedup vs baseline: 1.3291x; 1.3291x over previous
import jax
import jax.numpy as jnp
from jax.experimental import pallas as pl
from jax.experimental.pallas import tpu as pltpu

BN_EPS = 1e-5

_BN1_R = 0
_BN2_R = 2
_W2T_R = 4
_W3A_R = 68
_W3B_R = 132
_P_ROWS = 148


def _fused_kernel(x_ref, ohT_ref, w1_ref, p_ref, out_ref,
                  h1_ref, h2_ref, s1_ref, s2_ref):
    step = pl.program_id(0)
    nblk = pl.num_programs(0) // 3
    blk = x_ref.shape[0]
    b_total = h1_ref.shape[0]
    inv_b = 1.0 / b_total

    @pl.when(step < nblk)
    def _phase_a():
        xb = x_ref[...].astype(jnp.bfloat16)
        h1 = jnp.dot(xb, w1_ref[...].astype(jnp.bfloat16),
                     preferred_element_type=jnp.float32)
        h1_ref[pl.ds(step * blk, blk), :] = h1.astype(jnp.bfloat16)

        @pl.when(step == 0)
        def _():
            s1_ref[...] = jnp.zeros_like(s1_ref)

        s1_ref[...] += jnp.stack([jnp.sum(h1, axis=0),
                                  jnp.sum(h1 * h1, axis=0)])

    @pl.when((step >= nblk) & (step < 2 * nblk))
    def _phase_b():
        i = step - nblk
        totals = s1_ref[...]
        mean = totals[0:1, :] * inv_b
        var = totals[1:2, :] * inv_b - mean * mean
        scale = p_ref[_BN1_R:_BN1_R + 1, :] * jax.lax.rsqrt(var + BN_EPS)
        shift = p_ref[_BN1_R + 1:_BN1_R + 2, :] - mean * scale
        h1 = h1_ref[pl.ds(i * blk, blk), :].astype(jnp.float32)
        h1n = jnp.maximum(h1 * scale + shift, 0.0)
        w2t = p_ref[_W2T_R:_W2T_R + 64, :].astype(jnp.bfloat16)
        h2 = jax.lax.dot_general(h1n.astype(jnp.bfloat16), w2t,
                                 (((1,), (1,)), ((), ())),
                                 preferred_element_type=jnp.float32)
        h2_ref[pl.ds(i * blk, blk), :] = h2.astype(jnp.bfloat16)

        @pl.when(i == 0)
        def _():
            s2_ref[...] = jnp.zeros_like(s2_ref)

        s2_ref[...] += jnp.stack([jnp.sum(h2, axis=0),
                                  jnp.sum(h2 * h2, axis=0)])

    @pl.when(step >= 2 * nblk)
    def _phase_c():
        i = step - 2 * nblk
        totals = s2_ref[...]
        mean = totals[0:1, :] * inv_b
        var = totals[1:2, :] * inv_b - mean * mean
        scale = p_ref[_BN2_R:_BN2_R + 1, 0:64] * jax.lax.rsqrt(var + BN_EPS)
        shift = p_ref[_BN2_R + 1:_BN2_R + 2, 0:64] - mean * scale
        h2 = h2_ref[pl.ds(i * blk, blk), :].astype(jnp.float32)
        h2n = jnp.maximum(h2 * scale + shift, 0.0)
        out_dim = out_ref.shape[1]
        w3a = p_ref[_W3A_R:_W3A_R + 64, 0:out_dim].astype(jnp.bfloat16)
        w3b = p_ref[_W3B_R:_W3B_R + 16, 0:out_dim].astype(jnp.bfloat16)
        oht = ohT_ref[...].astype(jnp.bfloat16)
        logits = (jnp.dot(h2n.astype(jnp.bfloat16), w3a,
                          preferred_element_type=jnp.float32)
                  + jax.lax.dot_general(oht, w3b, (((0,), (0,)), ((), ())),
                                        preferred_element_type=jnp.float32))
        out_ref[...] = jax.nn.sigmoid(logits)


def kernel(latent_vec, class_onehot, w1, bn1, w2, bn2, w3a_pad, w3b_pad,
           b3_pad, output_dim=3):
    B, latent_dim = latent_vec.shape
    H0 = w1.shape[1]
    H1 = w2.shape[1]
    OUTP = w3a_pad.shape[1]
    C = class_onehot.shape[1]

    blk = 2048 if B % 2048 == 0 else B
    nblk = B // blk
    nsteps = 3 * nblk

    oht = jnp.transpose(class_onehot)
    w2t = jnp.transpose(w2)
    packed = jnp.concatenate([
        bn1,
        jnp.pad(bn2, ((0, 0), (0, H0 - H1))),
        w2t,
        jnp.pad(w3a_pad, ((0, 0), (0, H0 - OUTP))),
        jnp.pad(w3b_pad + b3_pad, ((0, 0), (0, H0 - OUTP))),
    ], axis=0)

    flops = (2 * B * (latent_dim * H0 + H0 * H1 + H1 * output_dim
                      + C * output_dim) + 12 * B * (H0 + H1))
    bytes_accessed = (B * latent_dim * 4 + B * C * 4
                      + (latent_dim + _P_ROWS) * H0 * 4
                      + B * output_dim * 4)

    grid_spec = pltpu.PrefetchScalarGridSpec(
        num_scalar_prefetch=0,
        grid=(nsteps,),
        in_specs=[
            pl.BlockSpec((blk, latent_dim),
                         lambda s, n=nblk: (jnp.minimum(s, n - 1), 0)),
            pl.BlockSpec((C, blk),
                         lambda s, n=nblk: (0, jnp.maximum(s - 2 * n, 0))),
            pl.BlockSpec(memory_space=pltpu.MemorySpace.VMEM),
            pl.BlockSpec(memory_space=pltpu.MemorySpace.VMEM),
        ],
        out_specs=pl.BlockSpec(
            (blk, output_dim),
            lambda s, n=nblk: (jnp.maximum(s - 2 * n, 0), 0)),
        scratch_shapes=[
            pltpu.VMEM((B, H0), jnp.bfloat16),
            pltpu.VMEM((B, H1), jnp.bfloat16),
            pltpu.VMEM((2, H0), jnp.float32),
            pltpu.VMEM((2, H1), jnp.float32),
        ],
    )

    return pl.pallas_call(
        _fused_kernel,
        out_shape=jax.ShapeDtypeStruct((B, output_dim), jnp.float32),
        grid_spec=grid_spec,
        compiler_params=pltpu.CompilerParams(
            dimension_semantics=("arbitrary",),
            vmem_limit_bytes=48 * 1024 * 1024),
        cost_estimate=pl.CostEstimate(
            flops=flops,
            transcendentals=B * output_dim + H0 + H1,
            bytes_accessed=bytes_accessed),
    )(latent_vec, oht, w1, packed)


# speedup vs baseline: 1.4082x; 1.0595x over previous
import jax
import jax.numpy as jnp
from jax.experimental import pallas as pl
from jax.experimental.pallas import tpu as pltpu

BN_EPS = 1e-5

_BN1_R = 0
_W2T_R = 2
_W3A_R = 0
_W3B_R = 64


def _fused_kernel(x_ref, ohT_ref, w1_ref, p256_ref, p128_ref, bn2_ref,
                  out_ref, h1_ref, h2_ref, s1_ref, s2_ref):
    step = pl.program_id(0)
    nblk = pl.num_programs(0) // 3
    blk = x_ref.shape[0]
    b_total = h1_ref.shape[0]
    inv_b = 1.0 / b_total

    @pl.when(step < nblk)
    def _phase_a():
        xb = x_ref[...].astype(jnp.bfloat16)
        h1 = jnp.dot(xb, w1_ref[...].astype(jnp.bfloat16),
                     preferred_element_type=jnp.float32)
        h1_ref[pl.ds(step * blk, blk), :] = h1.astype(jnp.bfloat16)

        @pl.when(step == 0)
        def _():
            s1_ref[...] = jnp.zeros_like(s1_ref)

        s1_ref[...] += jnp.stack([jnp.sum(h1, axis=0),
                                  jnp.sum(h1 * h1, axis=0)])

    @pl.when((step >= nblk) & (step < 2 * nblk))
    def _phase_b():
        i = step - nblk
        totals = s1_ref[...]
        mean = totals[0:1, :] * inv_b
        var = totals[1:2, :] * inv_b - mean * mean
        scale = p256_ref[_BN1_R:_BN1_R + 1, :] * jax.lax.rsqrt(var + BN_EPS)
        shift = p256_ref[_BN1_R + 1:_BN1_R + 2, :] - mean * scale
        h1 = h1_ref[pl.ds(i * blk, blk), :].astype(jnp.float32)
        h1n = jnp.maximum(h1 * scale + shift, 0.0)
        w2t = p256_ref[_W2T_R:_W2T_R + 64, :].astype(jnp.bfloat16)
        h2 = jax.lax.dot_general(h1n.astype(jnp.bfloat16), w2t,
                                 (((1,), (1,)), ((), ())),
                                 preferred_element_type=jnp.float32)
        h2_ref[pl.ds(i * blk, blk), :] = h2.astype(jnp.bfloat16)

        @pl.when(i == 0)
        def _():
            s2_ref[...] = jnp.zeros_like(s2_ref)

        s2_ref[...] += jnp.stack([jnp.sum(h2, axis=0),
                                  jnp.sum(h2 * h2, axis=0)])

    @pl.when(step >= 2 * nblk)
    def _phase_c():
        i = step - 2 * nblk
        totals = s2_ref[...]
        mean = totals[0:1, :] * inv_b
        var = totals[1:2, :] * inv_b - mean * mean
        scale = bn2_ref[0:1, :] * jax.lax.rsqrt(var + BN_EPS)
        shift = bn2_ref[1:2, :] - mean * scale
        h2 = h2_ref[pl.ds(i * blk, blk), :].astype(jnp.float32)
        h2n = jnp.maximum(h2 * scale + shift, 0.0)
        out_dim = out_ref.shape[1]
        w3a = p128_ref[_W3A_R:_W3A_R + 64, 0:out_dim].astype(jnp.bfloat16)
        w3b = p128_ref[_W3B_R:_W3B_R + 16, 0:out_dim].astype(jnp.bfloat16)
        oht = ohT_ref[:, pl.ds(i * blk, blk)].astype(jnp.bfloat16)
        logits = (jnp.dot(h2n.astype(jnp.bfloat16), w3a,
                          preferred_element_type=jnp.float32)
                  + jax.lax.dot_general(oht, w3b, (((0,), (0,)), ((), ())),
                                        preferred_element_type=jnp.float32))
        out_ref[...] = jax.nn.sigmoid(logits)


def kernel(latent_vec, class_onehot, w1, bn1, w2, bn2, w3a_pad, w3b_pad,
           b3_pad, output_dim=3):
    B, latent_dim = latent_vec.shape
    H0 = w1.shape[1]
    H1 = w2.shape[1]
    OUTP = w3a_pad.shape[1]
    C = class_onehot.shape[1]

    blk = 2048 if B % 2048 == 0 else B
    nblk = B // blk
    nsteps = 3 * nblk

    oht = jnp.transpose(class_onehot)
    w2t = jnp.transpose(w2)
    p256 = jnp.concatenate([bn1, w2t], axis=0)
    p128 = jnp.concatenate([w3a_pad, w3b_pad + b3_pad], axis=0)

    flops = (2 * B * (latent_dim * H0 + H0 * H1 + H1 * output_dim
                      + C * output_dim) + 12 * B * (H0 + H1))
    bytes_accessed = (B * latent_dim * 4 + B * C * 4
                      + (latent_dim + 2 + H1) * H0 * 4 + (H1 + C) * OUTP * 4
                      + B * output_dim * 4)

    grid_spec = pltpu.PrefetchScalarGridSpec(
        num_scalar_prefetch=0,
        grid=(nsteps,),
        in_specs=[
            pl.BlockSpec((blk, latent_dim),
                         lambda s, n=nblk: (jnp.minimum(s, n - 1), 0)),
            pl.BlockSpec((C, B), lambda s: (0, 0)),
            pl.BlockSpec((latent_dim, H0), lambda s: (0, 0)),
            pl.BlockSpec((2 + H1, H0), lambda s: (0, 0)),
            pl.BlockSpec((H1 + C, OUTP), lambda s: (0, 0)),
            pl.BlockSpec((2, H1), lambda s: (0, 0)),
        ],
        out_specs=pl.BlockSpec(
            (blk, output_dim),
            lambda s, n=nblk: (jnp.maximum(s - 2 * n, 0), 0)),
        scratch_shapes=[
            pltpu.VMEM((B, H0), jnp.bfloat16),
            pltpu.VMEM((B, H1), jnp.bfloat16),
            pltpu.VMEM((2, H0), jnp.float32),
            pltpu.VMEM((2, H1), jnp.float32),
        ],
    )

    return pl.pallas_call(
        _fused_kernel,
        out_shape=jax.ShapeDtypeStruct((B, output_dim), jnp.float32),
        grid_spec=grid_spec,
        compiler_params=pltpu.CompilerParams(
            dimension_semantics=("arbitrary",),
            vmem_limit_bytes=48 * 1024 * 1024),
        cost_estimate=pl.CostEstimate(
            flops=flops,
            transcendentals=B * output_dim + H0 + H1,
            bytes_accessed=bytes_accessed),
    )(latent_vec, oht, w1, p256, p128, bn2)
